# baseline (speedup 1.0000x reference)
import functools

import jax
import jax.numpy as jnp
from jax import lax
from jax.experimental import pallas as pl
from jax.experimental.pallas import tpu as pltpu
from jax.experimental.pallas import tpu_sc as plsc

_INFO = plsc.get_sparse_core_info()
_NC, _NS = _INFO.num_cores, _INFO.num_subcores
_NW = _NC * _NS


def _make_sc_gather(B, D):
    bpw = B // _NW
    mesh = plsc.VectorSubcoreMesh(core_axis_name="c", subcore_axis_name="s")

    G = 16
    W = 128

    H = 8
    NS_ = bpw // H

    @functools.partial(
        pl.kernel,
        mesh=mesh,
        out_type=[
            jax.ShapeDtypeStruct((D, B), jnp.float32),
            jax.ShapeDtypeStruct((D, B), jnp.float32),
        ],
        scratch_types=[
            pltpu.VMEM((2, bpw + G), jnp.int32),
            pltpu.VMEM((D, 2 * H * W), jnp.float32),
            pltpu.VMEM((D, bpw + G), jnp.float32),
            pltpu.VMEM((D, bpw + G), jnp.float32),
            pltpu.SemaphoreType.DMA,
            pltpu.SemaphoreType.DMA,
        ],
        compiler_params=pltpu.CompilerParams(
            use_tc_tiling_on_sc=True, needs_layout_passes=False),
    )
    def gather(uids, iids, utabT, itabT, u_outT, i_outT,
               ids_vm, slab, uoutT, ioutT, semA, semB):
        wid = lax.axis_index("s") * _NC + lax.axis_index("c")
        base = wid * bpw
        pltpu.sync_copy(uids.at[pl.ds(base, bpw)], ids_vm.at[0, pl.ds(0, bpw)])
        pltpu.sync_copy(iids.at[pl.ds(base, bpw)], ids_vm.at[1, pl.ds(0, bpw)])
        lane = lax.iota(jnp.int32, G)
        sems = (semA, semB)

        def issue(t, tabT, s, h, sem):
            vec = ids_vm[t, pl.ds(s * H, G)]
            for l in range(H):
                tid = jnp.sum(jnp.where(lane == l, vec, 0))
                off = pl.multiple_of(tid & ~(W - 1), W)
                pltpu.make_async_copy(
                    tabT.at[:, pl.ds(off, W)],
                    slab.at[:, pl.ds((h * H + l) * W, W)], sem).start()

        def drain(tabT, h, sem):
            pltpu.make_async_copy(
                tabT.at[:, pl.ds(0, H * W)],
                slab.at[:, pl.ds(h * H * W, H * W)], sem).wait()

        def extract(t, outT, s, h):
            vec = ids_vm[t, pl.ds(s * H, G)]
            idx1 = (lane & (H - 1)) * W + h * (H * W) + (vec & (W - 1))
            lo = lane < H
            for c in range(D):
                idx0 = jnp.broadcast_to(jnp.int32(c), (G,))
                row = plsc.load_gather(slab, [idx0, idx1])
                plsc.store_compressed(
                    outT.at[c, pl.ds(s * H, G)], row, mask=lo)

        def make_pass(t, tabT, outT):
            issue(t, tabT, 0, 0, semA)

            def pair(g, _):
                s = g * 2
                issue(t, tabT, s + 1, 1, semB)
                drain(tabT, 0, semA)
                extract(t, outT, s, 0)

                @pl.when(g < NS_ // 2 - 1)
                def _():
                    issue(t, tabT, s + 2, 0, semA)

                drain(tabT, 1, semB)
                extract(t, outT, s + 1, 1)
                return 0

            lax.fori_loop(0, NS_ // 2, pair, 0)

        make_pass(0, utabT, uoutT)
        make_pass(1, itabT, ioutT)
        pltpu.sync_copy(uoutT.at[:, pl.ds(0, bpw)],
                        u_outT.at[:, pl.ds(base, bpw)])
        pltpu.sync_copy(ioutT.at[:, pl.ds(0, bpw)],
                        i_outT.at[:, pl.ds(base, bpw)])

    return gather


def _mlp_body(xuT, xiT, w1aT, w1bT, b1c, w2T, b2c, w3T, b3c, outT):
    h = jnp.dot(w1aT[...], xuT[...], preferred_element_type=jnp.float32)
    h = h + jnp.dot(w1bT[...], xiT[...], preferred_element_type=jnp.float32)
    h = jnp.maximum(h + b1c[...], 0.0)
    h = jnp.maximum(
        jnp.dot(w2T[...], h, preferred_element_type=jnp.float32) + b2c[...],
        0.0)
    outT[...] = (
        jnp.dot(w3T[...], h, preferred_element_type=jnp.float32) + b3c[...])


def _mlp_tc(xuT, xiT, w1aT, w1bT, b1c, w2T, b2c, w3T, b3c, blk=2048):
    D, B = xuT.shape
    H1 = w1aT.shape[0]
    H2 = w2T.shape[0]
    grid = (B // blk,)
    full = lambda shape: pl.BlockSpec(shape, lambda i: (0, 0))
    return pl.pallas_call(
        _mlp_body,
        grid=grid,
        in_specs=[
            pl.BlockSpec((D, blk), lambda i: (0, i)),
            pl.BlockSpec((D, blk), lambda i: (0, i)),
            full((H1, D)), full((H1, D)), full((H1, 1)),
            full((H2, H1)), full((H2, 1)),
            full((1, H2)), full((1, 1)),
        ],
        out_specs=pl.BlockSpec((1, blk), lambda i: (0, i)),
        out_shape=jax.ShapeDtypeStruct((1, B), jnp.float32),
    )(xuT, xiT, w1aT, w1bT, b1c, w2T, b2c, w3T, b3c)


def kernel(user_ids, item_ids, user_table, item_table, W1, b1, W2, b2, W3, b3):
    B = user_ids.shape[0]
    D = user_table.shape[1]
    uids = user_ids.astype(jnp.int32)
    iids = item_ids.astype(jnp.int32)
    u_embT, i_embT = _make_sc_gather(B, D)(
        uids, iids, user_table.T, item_table.T)
    W1T = W1.T
    outT = _mlp_tc(u_embT, i_embT, W1T[:, :D], W1T[:, D:], b1.reshape(-1, 1),
                   W2.T, b2.reshape(-1, 1), W3.T, b3.reshape(1, 1))
    return outT.reshape(B)
